# baseline (speedup 1.0000x reference)
import functools

import jax
import jax.numpy as jnp
from jax.experimental import pallas as pl
from jax.experimental.pallas import tpu as pltpu

Q_ROWS = 256
K_EMB = 50000
DIM = 1024
TOP = 6
KB = 2000
NB = K_EMB // KB
KSPLIT = KB // 2

BIGF = 3e38


def _knn_body(q_ref, e1_ref, e2_ref, x2_ref, y2_ref, outv_ref, outi_ref,
              xy_ref, rv_ref, ri_ref):
    b = pl.program_id(0)

    @pl.when(b < NB)
    def _gemm_a():
        xy_ref[b % 2, :, :KSPLIT] = jax.lax.dot_general(
            q_ref[...], e1_ref[...], (((1,), (1,)), ((), ())),
            preferred_element_type=jnp.float32)

    xy = xy_ref[(b + 1) % 2]
    x2 = x2_ref[...]
    y2 = y2_ref[0]
    d = jnp.sqrt(jnp.maximum(x2 + y2 - 2.0 * xy, 0.0))

    col = (jax.lax.broadcasted_iota(jnp.int32, (1, KB), 1)
           + (b - 1) * KB).astype(jnp.float32)

    bv, bi = [], []
    for _ in range(TOP // 2):
        m = jnp.min(d, axis=1, keepdims=True)
        idx = jnp.min(jnp.where(d == m, col, BIGF), axis=1, keepdims=True)
        d = jnp.where(col == idx, BIGF, d)
        bv.append(m)
        bi.append(idx)

    @pl.when(b < NB)
    def _gemm_b():
        xy_ref[b % 2, :, KSPLIT:] = jax.lax.dot_general(
            q_ref[...], e2_ref[...], (((1,), (1,)), ((), ())),
            preferred_element_type=jnp.float32)

    for _ in range(TOP - TOP // 2):
        m = jnp.min(d, axis=1, keepdims=True)
        idx = jnp.min(jnp.where(d == m, col, BIGF), axis=1, keepdims=True)
        d = jnp.where(col == idx, BIGF, d)
        bv.append(m)
        bi.append(idx)
    padf = jnp.full((Q_ROWS, 1), BIGF, jnp.float32)
    pkv = jnp.concatenate(bv + [padf, padf], axis=1)
    pki = jnp.concatenate(bi + [padf, padf], axis=1)

    cv = jnp.concatenate([rv_ref[...], pkv], axis=1)
    ci = jnp.concatenate([ri_ref[...], pki], axis=1)
    mv, mi = [], []
    for _ in range(TOP):
        m = jnp.min(cv, axis=1, keepdims=True)
        sel = jnp.min(jnp.where(cv == m, ci, BIGF), axis=1, keepdims=True)
        cv = jnp.where((cv == m) & (ci == sel), BIGF, cv)
        mv.append(m)
        mi.append(sel)
    newv = jnp.concatenate(mv + [padf, padf], axis=1)
    newi = jnp.concatenate(mi + [padf, padf], axis=1)
    rv_ref[...] = newv
    ri_ref[...] = newi

    @pl.when(b == 0)
    def _init():
        rv_ref[...] = jnp.full((Q_ROWS, 8), BIGF, jnp.float32)
        ri_ref[...] = jnp.full((Q_ROWS, 8), BIGF, jnp.float32)

    @pl.when(b == NB)
    def _finish():
        outv_ref[...] = newv
        outi_ref[...] = newi


@functools.partial(jax.jit, static_argnames=())
def kernel(queries, embeddings):
    x2 = jnp.sum(queries * queries, axis=1, keepdims=True)
    y2 = jnp.sum(embeddings * embeddings, axis=1)
    y2p = y2.reshape(NB, 1, KB)
    outv, outi = pl.pallas_call(
        _knn_body,
        grid=(NB + 1,),
        in_specs=[
            pl.BlockSpec((Q_ROWS, DIM), lambda b: (0, 0)),
            pl.BlockSpec((KSPLIT, DIM),
                         lambda b: (2 * jnp.minimum(b, NB - 1), 0)),
            pl.BlockSpec((KSPLIT, DIM),
                         lambda b: (2 * jnp.minimum(b, NB - 1) + 1, 0)),
            pl.BlockSpec((Q_ROWS, 1), lambda b: (0, 0)),
            pl.BlockSpec((1, 1, KB),
                         lambda b: (jnp.maximum(b - 1, 0), 0, 0)),
        ],
        out_specs=[
            pl.BlockSpec((Q_ROWS, 8), lambda b: (0, 0)),
            pl.BlockSpec((Q_ROWS, 8), lambda b: (0, 0)),
        ],
        out_shape=[
            jax.ShapeDtypeStruct((Q_ROWS, 8), jnp.float32),
            jax.ShapeDtypeStruct((Q_ROWS, 8), jnp.float32),
        ],
        scratch_shapes=[
            pltpu.VMEM((2, Q_ROWS, KB), jnp.float32),
            pltpu.VMEM((Q_ROWS, 8), jnp.float32),
            pltpu.VMEM((Q_ROWS, 8), jnp.float32),
        ],
        compiler_params=pltpu.CompilerParams(
            dimension_semantics=("arbitrary",)),
    )(queries, embeddings, embeddings, x2, y2p)
    out = jnp.concatenate([outi[:, :TOP], outv[:, :TOP]], axis=0).T
    return out


# speedup vs baseline: 1.1945x; 1.1945x over previous
import functools

import jax
import jax.numpy as jnp
from jax.experimental import pallas as pl
from jax.experimental.pallas import tpu as pltpu

Q_ROWS = 256
K_EMB = 50000
DIM = 1024
TOP = 6
KB = 2000
NB = K_EMB // KB
KSPLIT = KB // 2

BIGF = 3e38


def _knn_body(q_ref, e1_ref, e2_ref, x2_ref, outv_ref, outi_ref,
              xy_ref, y2s_ref, rv_ref, ri_ref):
    b = pl.program_id(0)

    @pl.when(b < NB)
    def _gemm_a():
        xy_ref[b % 2, :, :KSPLIT] = jax.lax.dot_general(
            q_ref[...], e1_ref[...], (((1,), (1,)), ((), ())),
            preferred_element_type=jnp.float32)

    @pl.when(b < NB)
    def _y2():
        s1 = jnp.sum(e1_ref[...] * e1_ref[...], axis=1)
        s2 = jnp.sum(e2_ref[...] * e2_ref[...], axis=1)
        y2s_ref[b % 2] = jnp.concatenate([s1, s2])[None, :]

    xy = xy_ref[(b + 1) % 2]
    x2 = x2_ref[...]
    y2 = y2s_ref[(b + 1) % 2]
    d = jnp.sqrt(jnp.maximum(x2 + y2 - 2.0 * xy, 0.0))

    col = (jax.lax.broadcasted_iota(jnp.int32, (1, KB), 1)
           + (b - 1) * KB).astype(jnp.float32)

    bv, bi = [], []
    for _ in range(TOP // 2):
        m = jnp.min(d, axis=1, keepdims=True)
        idx = jnp.min(jnp.where(d == m, col, BIGF), axis=1, keepdims=True)
        d = jnp.where(col == idx, BIGF, d)
        bv.append(m)
        bi.append(idx)

    @pl.when(b < NB)
    def _gemm_b():
        xy_ref[b % 2, :, KSPLIT:] = jax.lax.dot_general(
            q_ref[...], e2_ref[...], (((1,), (1,)), ((), ())),
            preferred_element_type=jnp.float32)

    for _ in range(TOP - TOP // 2):
        m = jnp.min(d, axis=1, keepdims=True)
        idx = jnp.min(jnp.where(d == m, col, BIGF), axis=1, keepdims=True)
        d = jnp.where(col == idx, BIGF, d)
        bv.append(m)
        bi.append(idx)
    padf = jnp.full((Q_ROWS, 1), BIGF, jnp.float32)
    pkv = jnp.concatenate(bv + [padf, padf], axis=1)
    pki = jnp.concatenate(bi + [padf, padf], axis=1)

    cv = jnp.concatenate([rv_ref[...], pkv], axis=1)
    ci = jnp.concatenate([ri_ref[...], pki], axis=1)
    mv, mi = [], []
    for _ in range(TOP):
        m = jnp.min(cv, axis=1, keepdims=True)
        sel = jnp.min(jnp.where(cv == m, ci, BIGF), axis=1, keepdims=True)
        cv = jnp.where((cv == m) & (ci == sel), BIGF, cv)
        mv.append(m)
        mi.append(sel)
    newv = jnp.concatenate(mv + [padf, padf], axis=1)
    newi = jnp.concatenate(mi + [padf, padf], axis=1)
    rv_ref[...] = newv
    ri_ref[...] = newi

    @pl.when(b == 0)
    def _init():
        rv_ref[...] = jnp.full((Q_ROWS, 8), BIGF, jnp.float32)
        ri_ref[...] = jnp.full((Q_ROWS, 8), BIGF, jnp.float32)

    @pl.when(b == NB)
    def _finish():
        outv_ref[...] = newv
        outi_ref[...] = newi


@functools.partial(jax.jit, static_argnames=())
def kernel(queries, embeddings):
    x2 = jnp.sum(queries * queries, axis=1, keepdims=True)
    outv, outi = pl.pallas_call(
        _knn_body,
        grid=(NB + 1,),
        in_specs=[
            pl.BlockSpec((Q_ROWS, DIM), lambda b: (0, 0)),
            pl.BlockSpec((KSPLIT, DIM),
                         lambda b: (2 * jnp.minimum(b, NB - 1), 0)),
            pl.BlockSpec((KSPLIT, DIM),
                         lambda b: (2 * jnp.minimum(b, NB - 1) + 1, 0)),
            pl.BlockSpec((Q_ROWS, 1), lambda b: (0, 0)),
        ],
        out_specs=[
            pl.BlockSpec((Q_ROWS, 8), lambda b: (0, 0)),
            pl.BlockSpec((Q_ROWS, 8), lambda b: (0, 0)),
        ],
        out_shape=[
            jax.ShapeDtypeStruct((Q_ROWS, 8), jnp.float32),
            jax.ShapeDtypeStruct((Q_ROWS, 8), jnp.float32),
        ],
        scratch_shapes=[
            pltpu.VMEM((2, Q_ROWS, KB), jnp.float32),
            pltpu.VMEM((2, 1, KB), jnp.float32),
            pltpu.VMEM((Q_ROWS, 8), jnp.float32),
            pltpu.VMEM((Q_ROWS, 8), jnp.float32),
        ],
        compiler_params=pltpu.CompilerParams(
            dimension_semantics=("arbitrary",)),
    )(queries, embeddings, embeddings, x2)
    out = jnp.concatenate([outi[:, :TOP], outv[:, :TOP]], axis=0).T
    return out
